# baseline (speedup 1.0000x reference)
import jax
import jax.numpy as jnp
from jax import lax
from jax.experimental import pallas as pl
from jax.experimental.pallas import tpu as pltpu
from jax.experimental.pallas import tpu_sc as plsc

NC = 2
NS = 16
NW = NC * NS

VOCAB_ = 1000000
D = 64
B_TOTAL = 4096 * 50
ROWS_PER_W = B_TOTAL // NW
CHUNK = 128
CHUNKS = ROWS_PER_W // CHUNK


def _gather_body(idx_hbm, table_hbm, out_hbm, idx_v, buf, gsem):
    wid = lax.axis_index("s") * NC + lax.axis_index("c")
    base = wid * ROWS_PER_W

    pltpu.sync_copy(idx_hbm.at[wid], idx_v)

    pltpu.async_copy(table_hbm.at[idx_v.at[0]], buf.at[0], gsem)

    def body(j, _):
        slot = lax.rem(j, 2)
        nslot = 1 - slot

        @pl.when(j < CHUNKS - 1)
        def _():
            pltpu.async_copy(table_hbm.at[idx_v.at[j + 1]], buf.at[nslot], gsem)

        pltpu.make_async_copy(table_hbm.at[idx_v.at[j]], buf.at[slot], gsem).wait()

        pltpu.sync_copy(buf.at[slot], out_hbm.at[pl.ds(base + j * CHUNK, CHUNK)])
        return 0

    lax.fori_loop(0, CHUNKS, body, 0)


def kernel(indices, table):
    idx3 = indices.reshape(NW, CHUNKS, CHUNK)

    mesh = plsc.VectorSubcoreMesh(core_axis_name="c", subcore_axis_name="s")
    out = pl.kernel(
        _gather_body,
        out_type=jax.ShapeDtypeStruct((B_TOTAL, D), jnp.float32),
        mesh=mesh,
        scratch_types=[
            pltpu.VMEM((CHUNKS, CHUNK), jnp.int32),
            pltpu.VMEM((2, CHUNK, D), jnp.float32),
            pltpu.SemaphoreType.DMA,
        ],
        compiler_params=pltpu.CompilerParams(use_tc_tiling_on_sc=False),
    )(idx3, table)

    return out.reshape(indices.shape[0], indices.shape[1], D)


# speedup vs baseline: 1.0126x; 1.0126x over previous
import jax
import jax.numpy as jnp
from jax import lax
from jax.experimental import pallas as pl
from jax.experimental.pallas import tpu as pltpu
from jax.experimental.pallas import tpu_sc as plsc

NC = 2
NS = 16
NW = NC * NS

D = 64
B_TOTAL = 4096 * 50
ROWS_PER_W = B_TOTAL // NW
K = 1
CHUNK = K * 128
CHUNKS = ROWS_PER_W // CHUNK
NBUF = 6
GAHEAD = NBUF - 2


def _gather_body(idx_hbm, table_hbm, out_hbm, idx_v, buf, gsem, ssem):
    wid = lax.axis_index("s") * NC + lax.axis_index("c")
    base = wid * ROWS_PER_W

    pltpu.sync_copy(idx_hbm.at[wid], idx_v)

    def fire(f):
        pltpu.async_copy(
            table_hbm.at[idx_v.at[lax.rem(f, CHUNKS)]],
            buf.at[lax.rem(f, NBUF)],
            gsem,
        )

    def wait_gather(j):
        pltpu.make_async_copy(
            table_hbm.at[idx_v.at[lax.rem(j, CHUNKS)]],
            buf.at[lax.rem(j, NBUF)],
            gsem,
        ).wait()

    def drain_store():
        pltpu.make_async_copy(
            buf.at[0], out_hbm.at[pl.ds(base, CHUNK)], ssem
        ).wait()

    for f in range(GAHEAD):
        fire(f)

    def body(j, _):
        wait_gather(j)
        pltpu.async_copy(
            buf.at[lax.rem(j, NBUF)],
            out_hbm.at[pl.ds(base + j * CHUNK, CHUNK)],
            ssem,
        )
        f = j + GAHEAD

        @pl.when(f < CHUNKS)
        def _():
            @pl.when(j >= 2)
            def _():
                drain_store()
            fire(f)

        return 0

    lax.fori_loop(0, CHUNKS, body, 0)

    for _ in range(NBUF):
        drain_store()


def kernel(indices, table):
    idx4 = indices.reshape(NW, CHUNKS, 128)

    mesh = plsc.VectorSubcoreMesh(core_axis_name="c", subcore_axis_name="s")
    out = pl.kernel(
        _gather_body,
        out_type=jax.ShapeDtypeStruct((B_TOTAL, D), jnp.float32),
        mesh=mesh,
        scratch_types=[
            pltpu.VMEM((CHUNKS, 128), jnp.int32),
            pltpu.VMEM((NBUF, CHUNK, D), jnp.float32),
            pltpu.SemaphoreType.DMA,
            pltpu.SemaphoreType.DMA,
        ],
        compiler_params=pltpu.CompilerParams(use_tc_tiling_on_sc=False),
    )(idx4, table)

    return out.reshape(indices.shape[0], indices.shape[1], D)


# speedup vs baseline: 1.1661x; 1.1515x over previous
import jax
import jax.numpy as jnp
from jax import lax
from jax.experimental import pallas as pl
from jax.experimental.pallas import tpu as pltpu
from jax.experimental.pallas import tpu_sc as plsc

NC = 2
NS = 16
NW = NC * NS

VOCAB = 1000000
D = 64
DP = 128
B_TOTAL = 4096 * 50
ROWS_PER_W = B_TOTAL // NW
CHUNK = 128
CHUNKS = ROWS_PER_W // CHUNK
NBUF = 4
GAHEAD = NBUF - 2

BLK = 2048


def _convert_body(tt_ref, out_ref):
    out_ref[:, 0:D] = tt_ref[...].T


def _convert_table(table):
    tt = jnp.transpose(table)
    grid = (VOCAB + BLK - 1) // BLK
    return pl.pallas_call(
        _convert_body,
        grid=(grid,),
        in_specs=[pl.BlockSpec((D, BLK), lambda i: (0, i))],
        out_specs=pl.BlockSpec((BLK, DP), lambda i: (i, 0)),
        out_shape=jax.ShapeDtypeStruct((VOCAB, DP), jnp.float32),
    )(tt)


def _gather_body(idx_hbm, table_hbm, out_hbm, idx_v, buf, gsem, ssem):
    wid = lax.axis_index("s") * NC + lax.axis_index("c")
    base = wid * ROWS_PER_W

    pltpu.sync_copy(idx_hbm.at[wid], idx_v)

    def fire(f):
        pltpu.async_copy(
            table_hbm.at[idx_v.at[lax.rem(f, CHUNKS)]],
            buf.at[lax.rem(f, NBUF)],
            gsem,
        )

    def wait_gather(j):
        pltpu.make_async_copy(
            table_hbm.at[idx_v.at[lax.rem(j, CHUNKS)]],
            buf.at[lax.rem(j, NBUF)],
            gsem,
        ).wait()

    def drain_store():
        pltpu.make_async_copy(
            buf.at[0, :, 0:D], out_hbm.at[pl.ds(base, CHUNK)], ssem
        ).wait()

    for f in range(GAHEAD):
        fire(f)

    def body(j, _):
        wait_gather(j)
        pltpu.async_copy(
            buf.at[lax.rem(j, NBUF), :, 0:D],
            out_hbm.at[pl.ds(base + j * CHUNK, CHUNK)],
            ssem,
        )
        f = j + GAHEAD

        @pl.when(f < CHUNKS)
        def _():
            @pl.when(j >= 2)
            def _():
                drain_store()
            fire(f)

        return 0

    lax.fori_loop(0, CHUNKS, body, 0)

    for _ in range(NBUF):
        drain_store()


def kernel(indices, table):
    table2 = _convert_table(table)
    idx3 = indices.reshape(NW, CHUNKS, CHUNK)

    mesh = plsc.VectorSubcoreMesh(core_axis_name="c", subcore_axis_name="s")
    out = pl.kernel(
        _gather_body,
        out_type=jax.ShapeDtypeStruct((B_TOTAL, D), jnp.float32),
        mesh=mesh,
        scratch_types=[
            pltpu.VMEM((CHUNKS, CHUNK), jnp.int32),
            pltpu.VMEM((NBUF, CHUNK, DP), jnp.float32),
            pltpu.SemaphoreType.DMA,
            pltpu.SemaphoreType.DMA,
        ],
        compiler_params=pltpu.CompilerParams(use_tc_tiling_on_sc=False),
    )(idx3, table2)

    return out.reshape(indices.shape[0], indices.shape[1], D)


# speedup vs baseline: 1.4255x; 1.2224x over previous
import jax
import jax.numpy as jnp
from jax import lax
from jax.experimental import pallas as pl
from jax.experimental.pallas import tpu as pltpu
from jax.experimental.pallas import tpu_sc as plsc

NC = 2
NS = 16
NW = NC * NS

VOCAB = 1000000
D = 64
DP = 128
B_TOTAL = 4096 * 50
ROWS_PER_W = B_TOTAL // NW
CHUNK = 128
CHUNKS = ROWS_PER_W // CHUNK
NBUF = 4
GAHEAD = NBUF - 2

BLK = 4096


def _convert_body(tt_ref, out_ref):
    out_ref[:, 0:D] = tt_ref[...].T


def _convert_table(table):
    tt = jnp.transpose(table)
    grid = (VOCAB + BLK - 1) // BLK
    return pl.pallas_call(
        _convert_body,
        grid=(grid,),
        in_specs=[pl.BlockSpec((D, BLK), lambda i: (0, i))],
        out_specs=pl.BlockSpec((BLK, DP), lambda i: (i, 0)),
        out_shape=jax.ShapeDtypeStruct((VOCAB, DP), jnp.float32),
    )(tt)


def _gather_body(idx_hbm, table_hbm, out_hbm, idx_v, buf, gsem, ssem):
    wid = lax.axis_index("s") * NC + lax.axis_index("c")
    base = wid * ROWS_PER_W

    pltpu.sync_copy(idx_hbm.at[wid], idx_v)

    def fire(f):
        pltpu.async_copy(
            table_hbm.at[idx_v.at[lax.rem(f, CHUNKS)]],
            buf.at[lax.rem(f, NBUF)],
            gsem,
        )

    def wait_gather(j):
        pltpu.make_async_copy(
            table_hbm.at[idx_v.at[lax.rem(j, CHUNKS)]],
            buf.at[lax.rem(j, NBUF)],
            gsem,
        ).wait()

    def drain_store():
        pltpu.make_async_copy(
            buf.at[0, :, 0:D], out_hbm.at[pl.ds(base, CHUNK)], ssem
        ).wait()

    for f in range(GAHEAD):
        fire(f)

    def body(j, _):
        wait_gather(j)
        pltpu.async_copy(
            buf.at[lax.rem(j, NBUF), :, 0:D],
            out_hbm.at[pl.ds(base + j * CHUNK, CHUNK)],
            ssem,
        )
        f = j + GAHEAD

        @pl.when(f < CHUNKS)
        def _():
            @pl.when(j >= 2)
            def _():
                drain_store()
            fire(f)

        return 0

    lax.fori_loop(0, CHUNKS, body, 0)

    for _ in range(NBUF):
        drain_store()


def kernel(indices, table):
    table2 = _convert_table(table)
    idx3 = indices.reshape(NW, CHUNKS, CHUNK)

    mesh = plsc.VectorSubcoreMesh(core_axis_name="c", subcore_axis_name="s")
    out = pl.kernel(
        _gather_body,
        out_type=jax.ShapeDtypeStruct((B_TOTAL, D), jnp.float32),
        mesh=mesh,
        scratch_types=[
            pltpu.VMEM((CHUNKS, CHUNK), jnp.int32),
            pltpu.VMEM((NBUF, CHUNK, DP), jnp.float32),
            pltpu.SemaphoreType.DMA,
            pltpu.SemaphoreType.DMA,
        ],
        compiler_params=pltpu.CompilerParams(use_tc_tiling_on_sc=False),
    )(idx3, table2)

    return out.reshape(indices.shape[0], indices.shape[1], D)


# speedup vs baseline: 1.6468x; 1.1553x over previous
import jax
import jax.numpy as jnp
from jax import lax
from jax.experimental import pallas as pl
from jax.experimental.pallas import tpu as pltpu
from jax.experimental.pallas import tpu_sc as plsc

NC = 2
NS = 16
NW = NC * NS

VOCAB = 1000000
D = 64
DP = 128
B_TOTAL = 4096 * 50
ROWS_PER_W = B_TOTAL // NW
CHUNK = 128
CHUNKS = ROWS_PER_W // CHUNK
NBUF = 4
GAHEAD = NBUF - 2

BLK = 8192


def _convert_body(tt_ref, out_ref):
    out_ref[:, 0:D] = tt_ref[...].T


def _convert_table(table):
    tt = jnp.transpose(table)
    grid = (VOCAB + BLK - 1) // BLK
    return pl.pallas_call(
        _convert_body,
        grid=(grid,),
        in_specs=[pl.BlockSpec((D, BLK), lambda i: (0, i))],
        out_specs=pl.BlockSpec((BLK, DP), lambda i: (i, 0)),
        out_shape=jax.ShapeDtypeStruct((VOCAB, DP), jnp.float32),
    )(tt)


def _gather_body(idx_hbm, table_hbm, out_hbm, idx_v, buf, gsem, ssem):
    wid = lax.axis_index("s") * NC + lax.axis_index("c")
    base = wid * ROWS_PER_W

    pltpu.sync_copy(idx_hbm.at[wid], idx_v)

    def fire(f):
        pltpu.async_copy(
            table_hbm.at[idx_v.at[lax.rem(f, CHUNKS)]],
            buf.at[lax.rem(f, NBUF)],
            gsem,
        )

    def wait_gather(j):
        pltpu.make_async_copy(
            table_hbm.at[idx_v.at[lax.rem(j, CHUNKS)]],
            buf.at[lax.rem(j, NBUF)],
            gsem,
        ).wait()

    def drain_store():
        pltpu.make_async_copy(
            buf.at[0, :, 0:D], out_hbm.at[pl.ds(base, CHUNK)], ssem
        ).wait()

    for f in range(GAHEAD):
        fire(f)

    def body(j, _):
        wait_gather(j)
        pltpu.async_copy(
            buf.at[lax.rem(j, NBUF), :, 0:D],
            out_hbm.at[pl.ds(base + j * CHUNK, CHUNK)],
            ssem,
        )
        f = j + GAHEAD

        @pl.when(f < CHUNKS)
        def _():
            @pl.when(j >= 2)
            def _():
                drain_store()
            fire(f)

        return 0

    lax.fori_loop(0, CHUNKS, body, 0)

    for _ in range(NBUF):
        drain_store()


def kernel(indices, table):
    table2 = _convert_table(table)
    idx3 = indices.reshape(NW, CHUNKS, CHUNK)

    mesh = plsc.VectorSubcoreMesh(core_axis_name="c", subcore_axis_name="s")
    out = pl.kernel(
        _gather_body,
        out_type=jax.ShapeDtypeStruct((B_TOTAL, D), jnp.float32),
        mesh=mesh,
        scratch_types=[
            pltpu.VMEM((CHUNKS, CHUNK), jnp.int32),
            pltpu.VMEM((NBUF, CHUNK, DP), jnp.float32),
            pltpu.SemaphoreType.DMA,
            pltpu.SemaphoreType.DMA,
        ],
        compiler_params=pltpu.CompilerParams(use_tc_tiling_on_sc=False),
    )(idx3, table2)

    return out.reshape(indices.shape[0], indices.shape[1], D)


# speedup vs baseline: 1.7180x; 1.0432x over previous
import jax
import jax.numpy as jnp
from jax import lax
from jax.experimental import pallas as pl
from jax.experimental.pallas import tpu as pltpu
from jax.experimental.pallas import tpu_sc as plsc

NC = 2
NS = 16
NW = NC * NS

VOCAB = 1000000
D = 64
DP = 128
B_TOTAL = 4096 * 50
ROWS_PER_W = B_TOTAL // NW
CHUNK = 128
CHUNKS = ROWS_PER_W // CHUNK
NBUF = 4
GAHEAD = NBUF - 2

BLK = 16384


def _convert_body(tt_ref, out_ref):
    out_ref[:, 0:D] = tt_ref[...].T


def _convert_table(table):
    tt = jnp.transpose(table)
    grid = (VOCAB + BLK - 1) // BLK
    return pl.pallas_call(
        _convert_body,
        grid=(grid,),
        in_specs=[pl.BlockSpec((D, BLK), lambda i: (0, i))],
        out_specs=pl.BlockSpec((BLK, DP), lambda i: (i, 0)),
        out_shape=jax.ShapeDtypeStruct((VOCAB, DP), jnp.float32),
    )(tt)


def _gather_body(idx_hbm, table_hbm, out_hbm, idx_v, buf, gsem, ssem):
    wid = lax.axis_index("s") * NC + lax.axis_index("c")
    base = wid * ROWS_PER_W

    pltpu.sync_copy(idx_hbm.at[wid], idx_v)

    def fire(f):
        pltpu.async_copy(
            table_hbm.at[idx_v.at[lax.rem(f, CHUNKS)]],
            buf.at[lax.rem(f, NBUF)],
            gsem,
        )

    def wait_gather(j):
        pltpu.make_async_copy(
            table_hbm.at[idx_v.at[lax.rem(j, CHUNKS)]],
            buf.at[lax.rem(j, NBUF)],
            gsem,
        ).wait()

    def drain_store():
        pltpu.make_async_copy(
            buf.at[0, :, 0:D], out_hbm.at[pl.ds(base, CHUNK)], ssem
        ).wait()

    for f in range(GAHEAD):
        fire(f)

    def body(j, _):
        wait_gather(j)
        pltpu.async_copy(
            buf.at[lax.rem(j, NBUF), :, 0:D],
            out_hbm.at[pl.ds(base + j * CHUNK, CHUNK)],
            ssem,
        )
        f = j + GAHEAD

        @pl.when(f < CHUNKS)
        def _():
            @pl.when(j >= 2)
            def _():
                drain_store()
            fire(f)

        return 0

    lax.fori_loop(0, CHUNKS, body, 0)

    for _ in range(NBUF):
        drain_store()


def kernel(indices, table):
    table2 = _convert_table(table)
    idx3 = indices.reshape(NW, CHUNKS, CHUNK)

    mesh = plsc.VectorSubcoreMesh(core_axis_name="c", subcore_axis_name="s")
    out = pl.kernel(
        _gather_body,
        out_type=jax.ShapeDtypeStruct((B_TOTAL, D), jnp.float32),
        mesh=mesh,
        scratch_types=[
            pltpu.VMEM((CHUNKS, CHUNK), jnp.int32),
            pltpu.VMEM((NBUF, CHUNK, DP), jnp.float32),
            pltpu.SemaphoreType.DMA,
            pltpu.SemaphoreType.DMA,
        ],
        compiler_params=pltpu.CompilerParams(use_tc_tiling_on_sc=False),
    )(idx3, table2)

    return out.reshape(indices.shape[0], indices.shape[1], D)


# speedup vs baseline: 1.8908x; 1.1006x over previous
import jax
import jax.numpy as jnp
from jax import lax
from jax.experimental import pallas as pl
from jax.experimental.pallas import tpu as pltpu
from jax.experimental.pallas import tpu_sc as plsc

NC = 2
NS = 16
NW = NC * NS

VOCAB = 1000000
D = 64
DP = 128
B_TOTAL = 4096 * 50
ROWS_PER_W = B_TOTAL // NW
CHUNK = 128
CHUNKS = ROWS_PER_W // CHUNK
NBUF = 4
GAHEAD = NBUF - 2
LANES = 16
IGROUPS = CHUNKS * CHUNK // LANES

HALF = 8192
K = 14
NBLK = (VOCAB + 2 * HALF - 1) // (2 * HALF)
VPAD = NBLK * 2 * HALF


def _convert_body(tt_ref, out_ref):
    x = tt_ref[...]
    out_ref[:, 0:D] = x[:, 0:HALF].T
    out_ref[:, D:DP] = x[:, HALF:2 * HALF].T


def _convert_table(table):
    tt = jnp.transpose(table)
    packed = pl.pallas_call(
        _convert_body,
        grid=(NBLK,),
        in_specs=[pl.BlockSpec((D, 2 * HALF), lambda i: (0, i))],
        out_specs=pl.BlockSpec((HALF, DP), lambda i: (i, 0)),
        out_shape=jax.ShapeDtypeStruct((NBLK * HALF, DP), jnp.float32),
    )(tt)
    return packed.reshape(VPAD, D)


def _gather_body(idx_hbm, table_hbm, out_hbm, idx_v, buf, gsem, ssem):
    wid = lax.axis_index("s") * NC + lax.axis_index("c")
    base = wid * ROWS_PER_W

    pltpu.sync_copy(idx_hbm.at[wid], idx_v)

    def remap(t, _):
        c = t // (CHUNK // LANES)
        g = t % (CHUNK // LANES)
        r = idx_v[c, pl.ds(g * LANES, LANES)]
        b = lax.shift_right_logical(r, K)
        u = lax.bitwise_and(r, 2 ** K - 1)
        hi = lax.shift_right_logical(u, K - 1)
        lo = lax.bitwise_and(u, 2 ** (K - 1) - 1)
        r2 = lax.bitwise_or(
            lax.shift_left(b, K),
            lax.bitwise_or(lax.shift_left(lo, 1), hi),
        )
        idx_v[c, pl.ds(g * LANES, LANES)] = r2
        return 0

    lax.fori_loop(0, IGROUPS, remap, 0)

    def fire(f):
        pltpu.async_copy(
            table_hbm.at[idx_v.at[lax.rem(f, CHUNKS)]],
            buf.at[lax.rem(f, NBUF)],
            gsem,
        )

    def wait_gather(j):
        pltpu.make_async_copy(
            table_hbm.at[idx_v.at[lax.rem(j, CHUNKS)]],
            buf.at[lax.rem(j, NBUF)],
            gsem,
        ).wait()

    def drain_store():
        pltpu.make_async_copy(
            buf.at[0], out_hbm.at[pl.ds(base, CHUNK)], ssem
        ).wait()

    for f in range(GAHEAD):
        fire(f)

    def body(j, _):
        wait_gather(j)
        pltpu.async_copy(
            buf.at[lax.rem(j, NBUF)],
            out_hbm.at[pl.ds(base + j * CHUNK, CHUNK)],
            ssem,
        )
        f = j + GAHEAD

        @pl.when(f < CHUNKS)
        def _():
            @pl.when(j >= 2)
            def _():
                drain_store()
            fire(f)

        return 0

    lax.fori_loop(0, CHUNKS, body, 0)

    for _ in range(NBUF):
        drain_store()


def kernel(indices, table):
    table2 = _convert_table(table)
    idx3 = indices.reshape(NW, CHUNKS, CHUNK)

    mesh = plsc.VectorSubcoreMesh(core_axis_name="c", subcore_axis_name="s")
    out = pl.kernel(
        _gather_body,
        out_type=jax.ShapeDtypeStruct((B_TOTAL, D), jnp.float32),
        mesh=mesh,
        scratch_types=[
            pltpu.VMEM((CHUNKS, CHUNK), jnp.int32),
            pltpu.VMEM((NBUF, CHUNK, D), jnp.float32),
            pltpu.SemaphoreType.DMA,
            pltpu.SemaphoreType.DMA,
        ],
        compiler_params=pltpu.CompilerParams(use_tc_tiling_on_sc=False),
    )(idx3, table2)

    return out.reshape(indices.shape[0], indices.shape[1], D)


# speedup vs baseline: 1.9700x; 1.0419x over previous
import jax
import jax.numpy as jnp
from jax import lax
from jax.experimental import pallas as pl
from jax.experimental.pallas import tpu as pltpu
from jax.experimental.pallas import tpu_sc as plsc

NC = 2
NS = 16
NW = NC * NS

VOCAB = 1000000
D = 64
DP = 128
B_TOTAL = 4096 * 50
ROWS_PER_W = B_TOTAL // NW
CHUNK = 128
CHUNKS = ROWS_PER_W // CHUNK
NBUF = 4
GAHEAD = NBUF - 2
LANES = 16
IGROUPS = CHUNKS * CHUNK // LANES

HALF = 16384
K = 15
NBLK = (VOCAB + 2 * HALF - 1) // (2 * HALF)
VPAD = NBLK * 2 * HALF


def _convert_body(tt_ref, out_ref):
    x = tt_ref[...]
    out_ref[:, 0:D] = x[:, 0:HALF].T
    out_ref[:, D:DP] = x[:, HALF:2 * HALF].T


def _convert_table(table):
    tt = jnp.transpose(table)
    packed = pl.pallas_call(
        _convert_body,
        grid=(NBLK,),
        in_specs=[pl.BlockSpec((D, 2 * HALF), lambda i: (0, i))],
        out_specs=pl.BlockSpec((HALF, DP), lambda i: (i, 0)),
        out_shape=jax.ShapeDtypeStruct((NBLK * HALF, DP), jnp.float32),
    )(tt)
    return packed.reshape(VPAD, D)


def _gather_body(idx_hbm, table_hbm, out_hbm, idx_v, buf, gsem, ssem):
    wid = lax.axis_index("s") * NC + lax.axis_index("c")
    base = wid * ROWS_PER_W

    pltpu.sync_copy(idx_hbm.at[wid], idx_v)

    def remap(t, _):
        c = t // (CHUNK // LANES)
        g = t % (CHUNK // LANES)
        r = idx_v[c, pl.ds(g * LANES, LANES)]
        b = lax.shift_right_logical(r, K)
        u = lax.bitwise_and(r, 2 ** K - 1)
        hi = lax.shift_right_logical(u, K - 1)
        lo = lax.bitwise_and(u, 2 ** (K - 1) - 1)
        r2 = lax.bitwise_or(
            lax.shift_left(b, K),
            lax.bitwise_or(lax.shift_left(lo, 1), hi),
        )
        idx_v[c, pl.ds(g * LANES, LANES)] = r2
        return 0

    lax.fori_loop(0, IGROUPS, remap, 0)

    def fire(f):
        pltpu.async_copy(
            table_hbm.at[idx_v.at[lax.rem(f, CHUNKS)]],
            buf.at[lax.rem(f, NBUF)],
            gsem,
        )

    def wait_gather(j):
        pltpu.make_async_copy(
            table_hbm.at[idx_v.at[lax.rem(j, CHUNKS)]],
            buf.at[lax.rem(j, NBUF)],
            gsem,
        ).wait()

    def drain_store():
        pltpu.make_async_copy(
            buf.at[0], out_hbm.at[pl.ds(base, CHUNK)], ssem
        ).wait()

    for f in range(GAHEAD):
        fire(f)

    def body(j, _):
        wait_gather(j)
        pltpu.async_copy(
            buf.at[lax.rem(j, NBUF)],
            out_hbm.at[pl.ds(base + j * CHUNK, CHUNK)],
            ssem,
        )
        f = j + GAHEAD

        @pl.when(f < CHUNKS)
        def _():
            @pl.when(j >= 2)
            def _():
                drain_store()
            fire(f)

        return 0

    lax.fori_loop(0, CHUNKS, body, 0)

    for _ in range(NBUF):
        drain_store()


def kernel(indices, table):
    table2 = _convert_table(table)
    idx3 = indices.reshape(NW, CHUNKS, CHUNK)

    mesh = plsc.VectorSubcoreMesh(core_axis_name="c", subcore_axis_name="s")
    out = pl.kernel(
        _gather_body,
        out_type=jax.ShapeDtypeStruct((B_TOTAL, D), jnp.float32),
        mesh=mesh,
        scratch_types=[
            pltpu.VMEM((CHUNKS, CHUNK), jnp.int32),
            pltpu.VMEM((NBUF, CHUNK, D), jnp.float32),
            pltpu.SemaphoreType.DMA,
            pltpu.SemaphoreType.DMA,
        ],
        compiler_params=pltpu.CompilerParams(use_tc_tiling_on_sc=False),
    )(idx3, table2)

    return out.reshape(indices.shape[0], indices.shape[1], D)


# speedup vs baseline: 1.9810x; 1.0056x over previous
import jax
import jax.numpy as jnp
from jax import lax
from jax.experimental import pallas as pl
from jax.experimental.pallas import tpu as pltpu
from jax.experimental.pallas import tpu_sc as plsc

NC = 2
NS = 16
NW = NC * NS

VOCAB = 1000000
D = 64
DP = 128
B_TOTAL = 4096 * 50
ROWS_PER_W = B_TOTAL // NW
CHUNK = 128
CHUNKS = ROWS_PER_W // CHUNK
NBUF = 6
GAHEAD = NBUF - 2
LANES = 16
IGROUPS = CHUNKS * CHUNK // LANES

HALF = 16384
K = 15
NBLK = (VOCAB + 2 * HALF - 1) // (2 * HALF)
VPAD = NBLK * 2 * HALF


def _convert_body(tt_ref, out_ref):
    x = tt_ref[...]
    out_ref[:, 0:D] = x[:, 0:HALF].T
    out_ref[:, D:DP] = x[:, HALF:2 * HALF].T


def _convert_table(table):
    tt = jnp.transpose(table)
    packed = pl.pallas_call(
        _convert_body,
        grid=(NBLK,),
        in_specs=[pl.BlockSpec((D, 2 * HALF), lambda i: (0, i))],
        out_specs=pl.BlockSpec((HALF, DP), lambda i: (i, 0)),
        out_shape=jax.ShapeDtypeStruct((NBLK * HALF, DP), jnp.float32),
    )(tt)
    return packed.reshape(VPAD, D)


def _gather_body(idx_hbm, table_hbm, out_hbm, idx_v, buf, gsem, ssem):
    wid = lax.axis_index("s") * NC + lax.axis_index("c")
    base = wid * ROWS_PER_W

    pltpu.sync_copy(idx_hbm.at[wid], idx_v)

    def remap(t, _):
        c = t // (CHUNK // LANES)
        g = t % (CHUNK // LANES)
        r = idx_v[c, pl.ds(g * LANES, LANES)]
        b = lax.shift_right_logical(r, K)
        u = lax.bitwise_and(r, 2 ** K - 1)
        hi = lax.shift_right_logical(u, K - 1)
        lo = lax.bitwise_and(u, 2 ** (K - 1) - 1)
        r2 = lax.bitwise_or(
            lax.shift_left(b, K),
            lax.bitwise_or(lax.shift_left(lo, 1), hi),
        )
        idx_v[c, pl.ds(g * LANES, LANES)] = r2
        return 0

    lax.fori_loop(0, IGROUPS, remap, 0)

    def fire(f):
        pltpu.async_copy(
            table_hbm.at[idx_v.at[lax.rem(f, CHUNKS)]],
            buf.at[lax.rem(f, NBUF)],
            gsem,
        )

    def wait_gather(j):
        pltpu.make_async_copy(
            table_hbm.at[idx_v.at[lax.rem(j, CHUNKS)]],
            buf.at[lax.rem(j, NBUF)],
            gsem,
        ).wait()

    def drain_store():
        pltpu.make_async_copy(
            buf.at[0], out_hbm.at[pl.ds(base, CHUNK)], ssem
        ).wait()

    for f in range(GAHEAD):
        fire(f)

    def body(j, _):
        wait_gather(j)
        pltpu.async_copy(
            buf.at[lax.rem(j, NBUF)],
            out_hbm.at[pl.ds(base + j * CHUNK, CHUNK)],
            ssem,
        )
        f = j + GAHEAD

        @pl.when(f < CHUNKS)
        def _():
            @pl.when(j >= 2)
            def _():
                drain_store()
            fire(f)

        return 0

    lax.fori_loop(0, CHUNKS, body, 0)

    for _ in range(NBUF):
        drain_store()


def kernel(indices, table):
    table2 = _convert_table(table)
    idx3 = indices.reshape(NW, CHUNKS, CHUNK)

    mesh = plsc.VectorSubcoreMesh(core_axis_name="c", subcore_axis_name="s")
    out = pl.kernel(
        _gather_body,
        out_type=jax.ShapeDtypeStruct((B_TOTAL, D), jnp.float32),
        mesh=mesh,
        scratch_types=[
            pltpu.VMEM((CHUNKS, CHUNK), jnp.int32),
            pltpu.VMEM((NBUF, CHUNK, D), jnp.float32),
            pltpu.SemaphoreType.DMA,
            pltpu.SemaphoreType.DMA,
        ],
        compiler_params=pltpu.CompilerParams(use_tc_tiling_on_sc=False),
    )(idx3, table2)

    return out.reshape(indices.shape[0], indices.shape[1], D)
